# baseline (speedup 1.0000x reference)
import functools

import jax
import jax.numpy as jnp
from jax import lax
from jax.experimental import pallas as pl
from jax.experimental.pallas import tpu as pltpu
from jax.experimental.pallas import tpu_sc as plsc

_N = 10000
_NP = 10240
_E = 160000
_D = 256
_C = 4
_W = 32
_PASSES = _D // (_C * _W)
_S = 2000
_NBLK = _S // 16
_NCHUNK = _E // _S
_RNG = _E // 16
_SEG = _NP // 16


def _make_spmm(weighted: bool, with_deg: bool):
    out_type = [jax.ShapeDtypeStruct((_D, _NP), jnp.float32)]
    if with_deg:
        out_type.append(jax.ShapeDtypeStruct((_NP,), jnp.float32))

    scratch = [
        pltpu.VMEM((_C * _NP,), jnp.float32),
        pltpu.VMEM((_C * _NP,), jnp.float32),
        pltpu.VMEM((_S,), jnp.int32),
        pltpu.VMEM((_S,), jnp.int32),
        pltpu.VMEM((_S,), jnp.int32),
        pltpu.VMEM((_S,), jnp.int32),
        pltpu.SemaphoreType.DMA,
        pltpu.SemaphoreType.DMA,
    ]
    if weighted:
        scratch += [
            pltpu.VMEM((_S,), jnp.float32),
            pltpu.VMEM((_S,), jnp.float32),
        ]
    if with_deg:
        scratch += [
            pltpu.VMEM((_NP,), jnp.float32),
            pltpu.VMEM_SHARED((16, _NP), jnp.float32),
            pltpu.VMEM((_SEG,), jnp.float32),
            pltpu.VMEM((_SEG,), jnp.float32),
        ]

    def body(*refs):
        it = iter(refs)
        xT = next(it)
        src = next(it)
        dst = next(it)
        wgt = next(it) if weighted else None
        aggT = next(it)
        deg_out = next(it) if with_deg else None
        xcols = next(it)
        acc = next(it)
        srcb0 = next(it)
        dstb0 = next(it)
        srcb1 = next(it)
        dstb1 = next(it)
        sem0 = next(it)
        sem1 = next(it)
        if weighted:
            wb0 = next(it)
            wb1 = next(it)
        else:
            wb0 = wb1 = None
        slots = ((srcb0, dstb0, wb0, sem0), (srcb1, dstb1, wb1, sem1))
        if with_deg:
            degv = next(it)
            shdegs = next(it)
            tmpv = next(it)
            dsumv = next(it)

        c = lax.axis_index("c")
        s = lax.axis_index("s")
        wid = s * 2 + c
        iota = lax.iota(jnp.int32, 16)
        ones = jnp.ones((16,), jnp.float32)
        zeros = jnp.zeros((16,), jnp.float32)

        if with_deg:
            def zdeg(i, carry):
                degv[pl.ds(i * 16, 16)] = zeros
                return carry
            lax.fori_loop(0, _NP // 16, zdeg, 0)

        for p in range(_PASSES):
            g = p * _W + wid
            for col in range(_C):
                pltpu.sync_copy(xT.at[_C * g + col],
                                xcols.at[pl.ds(col * _NP, _NP)])

            def zacc(i, carry):
                acc[pl.ds(i * 16, 16)] = zeros
                return carry
            lax.fori_loop(0, _C * _NP // 16, zacc, 0)

            do_deg = with_deg and p == 0
            lo = s * _RNG
            hi = lo + _RNG

            def fire(ci, slot):
                sb, db, wbx, sm = slot
                pltpu.async_copy(src.at[pl.ds(ci * _S, _S)], sb, sm)
                pltpu.async_copy(dst.at[pl.ds(ci * _S, _S)], db, sm)
                if weighted:
                    pltpu.async_copy(wgt.at[pl.ds(ci * _S, _S)], wbx, sm)

            def drain(slot):
                sb, db, wbx, sm = slot
                pltpu.make_async_copy(src.at[pl.ds(0, _S)], sb, sm).wait()
                pltpu.make_async_copy(dst.at[pl.ds(0, _S)], db, sm).wait()
                if weighted:
                    pltpu.make_async_copy(wgt.at[pl.ds(0, _S)], wbx, sm).wait()

            def process(ci, slot):
                sb, db, wbx, _ = slot

                @plsc.parallel_loop(0, _NBLK, 1, unroll=8)
                def blk(b):
                    s16 = sb[pl.ds(b * 16, 16)]
                    d16 = db[pl.ds(b * 16, 16)]
                    if weighted:
                        w16 = wbx[pl.ds(b * 16, 16)]
                    for col in range(_C):
                        v = plsc.load_gather(xcols, [s16 + (col * _NP)])
                        if weighted:
                            v = v * w16
                        plsc.addupdate_scatter(acc, [d16 + (col * _NP)], v)
                    if do_deg:
                        e = ci * _S + b * 16 + iota
                        m = (e >= lo) & (e < hi)
                        plsc.addupdate_scatter(degv, [d16], ones, mask=m)

            fire(0, slots[0])

            def chunk2(cj, carry):
                ci0 = cj * 2
                fire(ci0 + 1, slots[1])
                drain(slots[0])
                process(ci0, slots[0])

                @pl.when(ci0 + 2 < _NCHUNK)
                def _():
                    fire(ci0 + 2, slots[0])
                drain(slots[1])
                process(ci0 + 1, slots[1])
                return carry
            lax.fori_loop(0, _NCHUNK // 2, chunk2, 0)

            for col in range(_C):
                pltpu.sync_copy(acc.at[pl.ds(col * _NP, _NP)],
                                aggT.at[_C * g + col])

            if do_deg:
                pltpu.sync_copy(degv, shdegs.at[s])
                plsc.subcore_barrier()

                @pl.when(c == 0)
                def _():
                    base = s * _SEG

                    def zsum(i, carry):
                        dsumv[pl.ds(i * 16, 16)] = zeros
                        return carry
                    lax.fori_loop(0, _SEG // 16, zsum, 0)

                    def addrow(j, carry):
                        pltpu.sync_copy(shdegs.at[j, pl.ds(base, _SEG)], tmpv)

                        def addblk(bb, bcarry):
                            sl = pl.ds(bb * 16, 16)
                            dsumv[sl] = dsumv[sl] + tmpv[sl]
                            return bcarry
                        lax.fori_loop(0, _SEG // 16, addblk, 0)
                        return carry
                    lax.fori_loop(0, 16, addrow, 0)
                    pltpu.sync_copy(dsumv, deg_out.at[pl.ds(base, _SEG)])

    mesh = plsc.VectorSubcoreMesh(core_axis_name="c", subcore_axis_name="s",
                                  num_cores=2, num_subcores=16)
    return pl.kernel(
        body, out_type=out_type, mesh=mesh, scratch_types=scratch,
        compiler_params=pltpu.CompilerParams(needs_layout_passes=False))


_spmm_deg = _make_spmm(weighted=False, with_deg=True)
_spmm_w = _make_spmm(weighted=True, with_deg=False)

_BN = 512


def _make_dense(residual_relu: bool):
    def dbody(x_ref, aggT_ref, deg_ref, Ws_ref, Wn_ref, b_ref, out_ref):
        xb = x_ref[...]
        self_part = jnp.dot(xb, Ws_ref[...], preferred_element_type=jnp.float32)
        neigh = lax.dot_general(
            aggT_ref[...], Wn_ref[...], (((0,), (0,)), ((), ())),
            preferred_element_type=jnp.float32)
        rdeg = 1.0 / jnp.maximum(deg_ref[...], 1.0)
        o = self_part + b_ref[...] + rdeg * neigh
        if residual_relu:
            o = jnp.maximum(o + xb, 0.0)
        out_ref[...] = o

    return pl.pallas_call(
        dbody,
        grid=(_NP // _BN,),
        in_specs=[
            pl.BlockSpec((_BN, _D), lambda i: (i, 0)),
            pl.BlockSpec((_D, _BN), lambda i: (0, i)),
            pl.BlockSpec((_BN, 1), lambda i: (i, 0)),
            pl.BlockSpec((_D, _D), lambda i: (0, 0)),
            pl.BlockSpec((_D, _D), lambda i: (0, 0)),
            pl.BlockSpec((1, _D), lambda i: (0, 0)),
        ],
        out_specs=pl.BlockSpec((_BN, _D), lambda i: (i, 0)),
        out_shape=jax.ShapeDtypeStruct((_NP, _D), jnp.float32),
    )


_dense_rr = _make_dense(residual_relu=True)
_dense_plain = _make_dense(residual_relu=False)


def kernel(in_feat, edge_index, edge_weights, W_self0, b0, W_neigh0,
           W_self1, b1, W_neigh1):
    src = edge_index[0]
    dst = edge_index[1]
    xp = jnp.zeros((_NP, _D), jnp.float32).at[:_N].set(in_feat)
    xT = xp.T
    aggT1, deg = _spmm_deg(xT, src, dst)
    degc = deg.reshape(_NP, 1)
    h = _dense_rr(xp, aggT1, degc, W_self0, W_neigh0, b0.reshape(1, _D))
    hT = h.T
    [aggT2] = _spmm_w(hT, src, dst, edge_weights)
    out = _dense_plain(h, aggT2, degc, W_self1, W_neigh1, b1.reshape(1, _D))
    return out[:_N]


# speedup vs baseline: 1.1070x; 1.1070x over previous
import functools

import jax
import jax.numpy as jnp
from jax import lax
from jax.experimental import pallas as pl
from jax.experimental.pallas import tpu as pltpu
from jax.experimental.pallas import tpu_sc as plsc

_N = 10000
_NP = 10240
_E = 160000
_D = 256
_C = 4
_W = 32
_PASSES = _D // (_C * _W)
_S = 2000
_NBLK = _S // 16
_NCHUNK = _E // _S
_RNG = _E // 16
_SEG = _NP // 16


def _make_spmm(weighted: bool, with_deg: bool):
    out_type = [jax.ShapeDtypeStruct((_D, _NP), jnp.float32)]
    if with_deg:
        out_type.append(jax.ShapeDtypeStruct((_NP,), jnp.float32))

    scratch = [
        pltpu.VMEM((_C * _NP,), jnp.float32),
        pltpu.VMEM((_C * _NP,), jnp.float32),
        pltpu.VMEM((_S,), jnp.int32),
        pltpu.VMEM((_S,), jnp.int32),
        pltpu.VMEM((_S,), jnp.int32),
        pltpu.VMEM((_S,), jnp.int32),
        pltpu.SemaphoreType.DMA,
        pltpu.SemaphoreType.DMA,
    ]
    if weighted:
        scratch += [
            pltpu.VMEM((_S,), jnp.float32),
            pltpu.VMEM((_S,), jnp.float32),
        ]
    if with_deg:
        scratch += [
            pltpu.VMEM((_NP,), jnp.float32),
            pltpu.VMEM_SHARED((16, _NP), jnp.float32),
            pltpu.VMEM((_SEG,), jnp.float32),
            pltpu.VMEM((_SEG,), jnp.float32),
        ]

    def body(*refs):
        it = iter(refs)
        xT = next(it)
        src = next(it)
        dst = next(it)
        wgt = next(it) if weighted else None
        aggT = next(it)
        deg_out = next(it) if with_deg else None
        xcols = next(it)
        acc = next(it)
        srcb0 = next(it)
        dstb0 = next(it)
        srcb1 = next(it)
        dstb1 = next(it)
        sem0 = next(it)
        sem1 = next(it)
        if weighted:
            wb0 = next(it)
            wb1 = next(it)
        else:
            wb0 = wb1 = None
        slots = ((srcb0, dstb0, wb0, sem0), (srcb1, dstb1, wb1, sem1))
        if with_deg:
            degv = next(it)
            shdegs = next(it)
            tmpv = next(it)
            dsumv = next(it)

        c = lax.axis_index("c")
        s = lax.axis_index("s")
        wid = s * 2 + c
        iota = lax.iota(jnp.int32, 16)
        ones = jnp.ones((16,), jnp.float32)
        zeros = jnp.zeros((16,), jnp.float32)

        if with_deg:
            def zdeg(i, carry):
                degv[pl.ds(i * 16, 16)] = zeros
                return carry
            lax.fori_loop(0, _NP // 16, zdeg, 0)

        for p in range(_PASSES):
            g = p * _W + wid
            for col in range(_C):
                pltpu.sync_copy(xT.at[_C * g + col],
                                xcols.at[pl.ds(col * _NP, _NP)])

            def zacc(i, carry):
                acc[pl.ds(i * 16, 16)] = zeros
                return carry
            lax.fori_loop(0, _C * _NP // 16, zacc, 0)

            do_deg = with_deg and p == 0
            lo = s * _RNG
            hi = lo + _RNG

            def fire(ci, slot):
                sb, db, wbx, sm = slot
                pltpu.async_copy(src.at[pl.ds(ci * _S, _S)], sb, sm)
                pltpu.async_copy(dst.at[pl.ds(ci * _S, _S)], db, sm)
                if weighted:
                    pltpu.async_copy(wgt.at[pl.ds(ci * _S, _S)], wbx, sm)

            def drain(slot):
                sb, db, wbx, sm = slot
                pltpu.make_async_copy(src.at[pl.ds(0, _S)], sb, sm).wait()
                pltpu.make_async_copy(dst.at[pl.ds(0, _S)], db, sm).wait()
                if weighted:
                    pltpu.make_async_copy(wgt.at[pl.ds(0, _S)], wbx, sm).wait()

            def process(ci, slot):
                sb, db, wbx, _ = slot

                @plsc.parallel_loop(0, _NBLK, 1, unroll=4)
                def blk(b):
                    s16 = sb[pl.ds(b * 16, 16)]
                    d16 = db[pl.ds(b * 16, 16)]
                    if weighted:
                        w16 = wbx[pl.ds(b * 16, 16)]
                    for col in range(_C):
                        v = plsc.load_gather(xcols, [s16 + (col * _NP)])
                        if weighted:
                            v = v * w16
                        plsc.addupdate_scatter(acc, [d16 + (col * _NP)], v)
                    if do_deg:
                        e = ci * _S + b * 16 + iota
                        m = (e >= lo) & (e < hi)
                        plsc.addupdate_scatter(degv, [d16], ones, mask=m)

            fire(0, slots[0])

            def chunk2(cj, carry):
                ci0 = cj * 2
                fire(ci0 + 1, slots[1])
                drain(slots[0])
                process(ci0, slots[0])

                @pl.when(ci0 + 2 < _NCHUNK)
                def _():
                    fire(ci0 + 2, slots[0])
                drain(slots[1])
                process(ci0 + 1, slots[1])
                return carry
            lax.fori_loop(0, _NCHUNK // 2, chunk2, 0)

            for col in range(_C):
                pltpu.sync_copy(acc.at[pl.ds(col * _NP, _NP)],
                                aggT.at[_C * g + col])

            if do_deg:
                pltpu.sync_copy(degv, shdegs.at[s])
                plsc.subcore_barrier()

                @pl.when(c == 0)
                def _():
                    base = s * _SEG

                    def zsum(i, carry):
                        dsumv[pl.ds(i * 16, 16)] = zeros
                        return carry
                    lax.fori_loop(0, _SEG // 16, zsum, 0)

                    def addrow(j, carry):
                        pltpu.sync_copy(shdegs.at[j, pl.ds(base, _SEG)], tmpv)

                        def addblk(bb, bcarry):
                            sl = pl.ds(bb * 16, 16)
                            dsumv[sl] = dsumv[sl] + tmpv[sl]
                            return bcarry
                        lax.fori_loop(0, _SEG // 16, addblk, 0)
                        return carry
                    lax.fori_loop(0, 16, addrow, 0)
                    pltpu.sync_copy(dsumv, deg_out.at[pl.ds(base, _SEG)])

    mesh = plsc.VectorSubcoreMesh(core_axis_name="c", subcore_axis_name="s",
                                  num_cores=2, num_subcores=16)
    return pl.kernel(
        body, out_type=out_type, mesh=mesh, scratch_types=scratch,
        compiler_params=pltpu.CompilerParams(needs_layout_passes=False))


_spmm_deg = _make_spmm(weighted=False, with_deg=True)
_spmm_w = _make_spmm(weighted=True, with_deg=False)

_BN = 512


def _dense1_body(xT_ref, aggT_ref, deg_ref, Ws_ref, Wn_ref, b_ref, hT_ref):
    xTb = xT_ref[...]
    self_t = lax.dot_general(
        Ws_ref[...], xTb, (((0,), (0,)), ((), ())),
        preferred_element_type=jnp.float32)
    neigh_t = lax.dot_general(
        Wn_ref[...], aggT_ref[...], (((0,), (0,)), ((), ())),
        preferred_element_type=jnp.float32)
    rdeg = 1.0 / jnp.maximum(deg_ref[...], 1.0)
    o = xTb + self_t + b_ref[...] + rdeg * neigh_t
    hT_ref[...] = jnp.maximum(o, 0.0)


_dense1 = pl.pallas_call(
    _dense1_body,
    grid=(_NP // _BN,),
    in_specs=[
        pl.BlockSpec((_D, _BN), lambda i: (0, i)),
        pl.BlockSpec((_D, _BN), lambda i: (0, i)),
        pl.BlockSpec((1, _BN), lambda i: (0, i)),
        pl.BlockSpec((_D, _D), lambda i: (0, 0)),
        pl.BlockSpec((_D, _D), lambda i: (0, 0)),
        pl.BlockSpec((_D, 1), lambda i: (0, 0)),
    ],
    out_specs=pl.BlockSpec((_D, _BN), lambda i: (0, i)),
    out_shape=jax.ShapeDtypeStruct((_D, _NP), jnp.float32),
)


def _dense2_body(hT_ref, aggT_ref, deg_ref, Ws_ref, Wn_ref, b_ref, out_ref):
    self_part = lax.dot_general(
        hT_ref[...], Ws_ref[...], (((0,), (0,)), ((), ())),
        preferred_element_type=jnp.float32)
    neigh = lax.dot_general(
        aggT_ref[...], Wn_ref[...], (((0,), (0,)), ((), ())),
        preferred_element_type=jnp.float32)
    rdeg = 1.0 / jnp.maximum(deg_ref[...], 1.0)
    out_ref[...] = self_part + b_ref[...] + rdeg * neigh


_dense2 = pl.pallas_call(
    _dense2_body,
    grid=(_NP // _BN,),
    in_specs=[
        pl.BlockSpec((_D, _BN), lambda i: (0, i)),
        pl.BlockSpec((_D, _BN), lambda i: (0, i)),
        pl.BlockSpec((_BN, 1), lambda i: (i, 0)),
        pl.BlockSpec((_D, _D), lambda i: (0, 0)),
        pl.BlockSpec((_D, _D), lambda i: (0, 0)),
        pl.BlockSpec((1, _D), lambda i: (0, 0)),
    ],
    out_specs=pl.BlockSpec((_BN, _D), lambda i: (i, 0)),
    out_shape=jax.ShapeDtypeStruct((_N, _D), jnp.float32),
)


def kernel(in_feat, edge_index, edge_weights, W_self0, b0, W_neigh0,
           W_self1, b1, W_neigh1):
    src = edge_index[0]
    dst = edge_index[1]
    xT = jnp.pad(in_feat, ((0, _NP - _N), (0, 0))).T
    aggT1, deg = _spmm_deg(xT, src, dst)
    hT = _dense1(xT, aggT1, deg.reshape(1, _NP), W_self0, W_neigh0,
                 b0.reshape(_D, 1))
    [aggT2] = _spmm_w(hT, src, dst, edge_weights)
    out = _dense2(hT, aggT2, deg.reshape(_NP, 1), W_self1, W_neigh1,
                  b1.reshape(1, _D))
    return out


# speedup vs baseline: 1.3032x; 1.1772x over previous
import jax
import jax.numpy as jnp
import numpy as np
from jax import lax
from jax.experimental import pallas as pl
from jax.experimental.pallas import tpu as pltpu
from jax.experimental.pallas import tpu_sc as plsc

_N = 10000
_NP = 10240
_E = 160000
_D = 256
_PK = _D // 2
_RPT = 4
_W = 32
_SEG = _NP // 16

_HI_MASK = np.int32(-65536)


def _unpack_pair(p):
    lo = plsc.bitcast(lax.shift_left(p, 16), jnp.float32)
    hi = plsc.bitcast(lax.bitwise_and(p, _HI_MASK), jnp.float32)
    return lo, hi


def _make_spmm(weighted: bool):
    S = 800 if weighted else 1600
    nblk = S // 16
    nchunk = _E // S

    out_type = [jax.ShapeDtypeStruct((_D, _NP), jnp.float32)]
    scratch = [
        pltpu.VMEM((_RPT * _NP,), jnp.int32),
        pltpu.VMEM((2 * _RPT * _NP,), jnp.float32),
        pltpu.VMEM((S,), jnp.int32),
        pltpu.VMEM((S,), jnp.int32),
        pltpu.VMEM((S,), jnp.int32),
        pltpu.VMEM((S,), jnp.int32),
        pltpu.SemaphoreType.DMA,
        pltpu.SemaphoreType.DMA,
    ]
    if weighted:
        scratch += [
            pltpu.VMEM((S,), jnp.float32),
            pltpu.VMEM((S,), jnp.float32),
        ]

    def body(*refs):
        it = iter(refs)
        xpk_hbm = next(it)
        src = next(it)
        dst = next(it)
        wgt = next(it) if weighted else None
        aggT = next(it)
        xpk = next(it)
        acc = next(it)
        srcb0 = next(it)
        dstb0 = next(it)
        srcb1 = next(it)
        dstb1 = next(it)
        sem0 = next(it)
        sem1 = next(it)
        if weighted:
            wb0 = next(it)
            wb1 = next(it)
        else:
            wb0 = wb1 = None
        slots = ((srcb0, dstb0, wb0, sem0), (srcb1, dstb1, wb1, sem1))

        c = lax.axis_index("c")
        s = lax.axis_index("s")
        wid = s * 2 + c
        zeros = jnp.zeros((16,), jnp.float32)

        for r in range(_RPT):
            pltpu.sync_copy(xpk_hbm.at[_RPT * wid + r],
                            xpk.at[pl.ds(r * _NP, _NP)])

        def zacc(i, carry):
            acc[pl.ds(i * 16, 16)] = zeros
            return carry
        lax.fori_loop(0, 2 * _RPT * _NP // 16, zacc, 0)

        def fire(ci, slot):
            sb, db, wbx, sm = slot
            pltpu.async_copy(src.at[pl.ds(ci * S, S)], sb, sm)
            pltpu.async_copy(dst.at[pl.ds(ci * S, S)], db, sm)
            if weighted:
                pltpu.async_copy(wgt.at[pl.ds(ci * S, S)], wbx, sm)

        def drain(slot):
            sb, db, wbx, sm = slot
            pltpu.make_async_copy(src.at[pl.ds(0, S)], sb, sm).wait()
            pltpu.make_async_copy(dst.at[pl.ds(0, S)], db, sm).wait()
            if weighted:
                pltpu.make_async_copy(wgt.at[pl.ds(0, S)], wbx, sm).wait()

        def process(slot):
            sb, db, wbx, _ = slot

            @plsc.parallel_loop(0, nblk, 1, unroll=4)
            def blk(b):
                s16 = sb[pl.ds(b * 16, 16)]
                d16 = db[pl.ds(b * 16, 16)]
                if weighted:
                    w16 = wbx[pl.ds(b * 16, 16)]
                for r in range(_RPT):
                    p = plsc.load_gather(xpk, [s16 + (r * _NP)])
                    lo, hi = _unpack_pair(p)
                    if weighted:
                        lo = lo * w16
                        hi = hi * w16
                    plsc.addupdate_scatter(acc, [d16 + (r * _NP)], lo)
                    plsc.addupdate_scatter(acc, [d16 + ((r + _RPT) * _NP)], hi)

        fire(0, slots[0])

        def chunk2(cj, carry):
            ci0 = cj * 2
            fire(ci0 + 1, slots[1])
            drain(slots[0])
            process(slots[0])

            @pl.when(ci0 + 2 < nchunk)
            def _():
                fire(ci0 + 2, slots[0])
            drain(slots[1])
            process(slots[1])
            return carry
        lax.fori_loop(0, nchunk // 2, chunk2, 0)

        for r in range(_RPT):
            pltpu.sync_copy(acc.at[pl.ds(r * _NP, _NP)],
                            aggT.at[_RPT * wid + r])
            pltpu.sync_copy(acc.at[pl.ds((r + _RPT) * _NP, _NP)],
                            aggT.at[_PK + _RPT * wid + r])

    mesh = plsc.VectorSubcoreMesh(core_axis_name="c", subcore_axis_name="s",
                                  num_cores=2, num_subcores=16)
    return pl.kernel(
        body, out_type=out_type, mesh=mesh, scratch_types=scratch,
        compiler_params=pltpu.CompilerParams(needs_layout_passes=False))


_spmm_plain = _make_spmm(weighted=False)
_spmm_w = _make_spmm(weighted=True)

_SD = 2000
_DBLK = _SD // 16
_DCH = _E // _SD


def _deg_body(dst, deg_out, degv, db0, db1, sem0, sem1, shdegs, tmpv, dsumv):
    c = lax.axis_index("c")
    s = lax.axis_index("s")
    zeros = jnp.zeros((16,), jnp.float32)
    ones = jnp.ones((16,), jnp.float32)

    def zdeg(i, carry):
        degv[pl.ds(i * 16, 16)] = zeros
        return carry
    lax.fori_loop(0, _NP // 16, zdeg, 0)

    base_c = s * 5
    slots = ((db0, sem0), (db1, sem1))

    def fire(ci, slot):
        db, sm = slot
        pltpu.async_copy(dst.at[pl.ds(ci * _SD, _SD)], db, sm)

    def drain(slot):
        db, sm = slot
        pltpu.make_async_copy(dst.at[pl.ds(0, _SD)], db, sm).wait()

    def process(slot):
        db, _ = slot

        @plsc.parallel_loop(0, _DBLK, 1, unroll=4)
        def blk(b):
            d16 = db[pl.ds(b * 16, 16)]
            plsc.addupdate_scatter(degv, [d16], ones)

    fire(base_c, slots[0])
    for k in range(5):
        slot = slots[k % 2]
        nxt = slots[(k + 1) % 2]
        if k < 4:
            fire(base_c + k + 1, nxt)
        drain(slot)
        process(slot)

    pltpu.sync_copy(degv, shdegs.at[s])
    plsc.subcore_barrier()

    @pl.when(c == 0)
    def _():
        base = s * _SEG

        def zsum(i, carry):
            dsumv[pl.ds(i * 16, 16)] = zeros
            return carry
        lax.fori_loop(0, _SEG // 16, zsum, 0)

        def addrow(j, carry):
            pltpu.sync_copy(shdegs.at[j, pl.ds(base, _SEG)], tmpv)

            def addblk(bb, bcarry):
                sl = pl.ds(bb * 16, 16)
                dsumv[sl] = dsumv[sl] + tmpv[sl]
                return bcarry
            lax.fori_loop(0, _SEG // 16, addblk, 0)
            return carry
        lax.fori_loop(0, 16, addrow, 0)
        pltpu.sync_copy(dsumv, deg_out.at[pl.ds(base, _SEG)])


_deg_kernel = pl.kernel(
    _deg_body,
    out_type=[jax.ShapeDtypeStruct((_NP,), jnp.float32)],
    mesh=plsc.VectorSubcoreMesh(core_axis_name="c", subcore_axis_name="s",
                                num_cores=2, num_subcores=16),
    scratch_types=[
        pltpu.VMEM((_NP,), jnp.float32),
        pltpu.VMEM((_SD,), jnp.int32),
        pltpu.VMEM((_SD,), jnp.int32),
        pltpu.SemaphoreType.DMA,
        pltpu.SemaphoreType.DMA,
        pltpu.VMEM_SHARED((16, _NP), jnp.float32),
        pltpu.VMEM((_SEG,), jnp.float32),
        pltpu.VMEM((_SEG,), jnp.float32),
    ],
    compiler_params=pltpu.CompilerParams(needs_layout_passes=False))

_BN = 512


def _dense1_body(xT_ref, aggT_ref, deg_ref, Ws_ref, Wn_ref, b_ref,
                 hT_ref, hpk_ref):
    xTb = xT_ref[...]
    self_t = lax.dot_general(
        Ws_ref[...], xTb, (((0,), (0,)), ((), ())),
        preferred_element_type=jnp.float32)
    neigh_t = lax.dot_general(
        Wn_ref[...], aggT_ref[...], (((0,), (0,)), ((), ())),
        preferred_element_type=jnp.float32)
    rdeg = 1.0 / jnp.maximum(deg_ref[...], 1.0)
    o = jnp.maximum(xTb + self_t + b_ref[...] + rdeg * neigh_t, 0.0)
    hT_ref[...] = o
    ob = o.astype(jnp.bfloat16)
    lo = lax.bitcast_convert_type(ob[:_PK], jnp.uint16).astype(jnp.int32)
    hi = lax.bitcast_convert_type(ob[_PK:], jnp.uint16).astype(jnp.int32)
    hpk_ref[...] = lax.bitwise_or(lax.shift_left(hi, 16), lo)


_dense1 = pl.pallas_call(
    _dense1_body,
    grid=(_NP // _BN,),
    in_specs=[
        pl.BlockSpec((_D, _BN), lambda i: (0, i)),
        pl.BlockSpec((_D, _BN), lambda i: (0, i)),
        pl.BlockSpec((1, _BN), lambda i: (0, i)),
        pl.BlockSpec((_D, _D), lambda i: (0, 0)),
        pl.BlockSpec((_D, _D), lambda i: (0, 0)),
        pl.BlockSpec((_D, 1), lambda i: (0, 0)),
    ],
    out_specs=[
        pl.BlockSpec((_D, _BN), lambda i: (0, i)),
        pl.BlockSpec((_PK, _BN), lambda i: (0, i)),
    ],
    out_shape=[
        jax.ShapeDtypeStruct((_D, _NP), jnp.float32),
        jax.ShapeDtypeStruct((_PK, _NP), jnp.int32),
    ],
)


def _dense2_body(hT_ref, aggT_ref, deg_ref, Ws_ref, Wn_ref, b_ref, out_ref):
    self_part = lax.dot_general(
        hT_ref[...], Ws_ref[...], (((0,), (0,)), ((), ())),
        preferred_element_type=jnp.float32)
    neigh = lax.dot_general(
        aggT_ref[...], Wn_ref[...], (((0,), (0,)), ((), ())),
        preferred_element_type=jnp.float32)
    rdeg = 1.0 / jnp.maximum(deg_ref[...], 1.0)
    out_ref[...] = self_part + b_ref[...] + rdeg * neigh


_dense2 = pl.pallas_call(
    _dense2_body,
    grid=(_NP // _BN,),
    in_specs=[
        pl.BlockSpec((_D, _BN), lambda i: (0, i)),
        pl.BlockSpec((_D, _BN), lambda i: (0, i)),
        pl.BlockSpec((_BN, 1), lambda i: (i, 0)),
        pl.BlockSpec((_D, _D), lambda i: (0, 0)),
        pl.BlockSpec((_D, _D), lambda i: (0, 0)),
        pl.BlockSpec((1, _D), lambda i: (0, 0)),
    ],
    out_specs=pl.BlockSpec((_BN, _D), lambda i: (i, 0)),
    out_shape=jax.ShapeDtypeStruct((_N, _D), jnp.float32),
)


def _pack_pairs_T(mT):
    mb = mT.astype(jnp.bfloat16)
    lo = lax.bitcast_convert_type(mb[:_PK], jnp.uint16).astype(jnp.int32)
    hi = lax.bitcast_convert_type(mb[_PK:], jnp.uint16).astype(jnp.int32)
    return lax.bitwise_or(lax.shift_left(hi, 16), lo)


def kernel(in_feat, edge_index, edge_weights, W_self0, b0, W_neigh0,
           W_self1, b1, W_neigh1):
    src = edge_index[0]
    dst = edge_index[1]
    xT = jnp.pad(in_feat, ((0, _NP - _N), (0, 0))).T
    xpk = _pack_pairs_T(xT)
    [deg] = _deg_kernel(dst)
    [aggT1] = _spmm_plain(xpk, src, dst)
    hT, hpk = _dense1(xT, aggT1, deg.reshape(1, _NP), W_self0, W_neigh0,
                      b0.reshape(_D, 1))
    [aggT2] = _spmm_w(hpk, src, dst, edge_weights)
    out = _dense2(hT, aggT2, deg.reshape(_NP, 1), W_self1, W_neigh1,
                  b1.reshape(1, _D))
    return out
